# baseline (speedup 1.0000x reference)
import functools

import jax
import jax.numpy as jnp
from jax import lax
from jax.experimental import pallas as pl
from jax.experimental.pallas import tpu as pltpu
from jax.experimental.pallas import tpu_sc as plsc

_B = 16384
_E = 16
_NC = 2
_NS = 16
_NW = _NC * _NS
_BPW = _B // _NW
_CH = 128
_NCH = _BPW // _CH


def _sc_body(uidx_hbm, iidx_hbm, uemb_hbm, iemb_hbm, ubias_hbm, ibias_hbm,
             part_hbm, ubg_hbm, ibg_hbm,
             uidx_v, iidx_v, urows_v, irows_v, ub_v, ib_v, acc_v):
    wid = lax.axis_index("s") * _NC + lax.axis_index("c")
    base = wid * _BPW

    pltpu.sync_copy(uidx_hbm.at[pl.ds(base, _BPW)], uidx_v)
    pltpu.sync_copy(iidx_hbm.at[pl.ds(base, _BPW)], iidx_v)

    for j in range(_NCH):
        c = pl.ds(j * _CH, _CH)
        pltpu.sync_copy(uemb_hbm.at[uidx_v.at[c]], urows_v.at[c])
        pltpu.sync_copy(iemb_hbm.at[iidx_v.at[c]], irows_v.at[c])
        pltpu.sync_copy(ubias_hbm.at[uidx_v.at[c]], ub_v.at[c])
        pltpu.sync_copy(ibias_hbm.at[iidx_v.at[c]], ib_v.at[c])

    def fma(i, acc):
        return acc + urows_v[i] * irows_v[i]

    acc = lax.fori_loop(0, _BPW, fma, jnp.zeros((_E,), jnp.float32))
    acc_v[...] = acc
    pltpu.sync_copy(acc_v, part_hbm.at[wid])

    pltpu.sync_copy(ub_v, ubg_hbm.at[pl.ds(base, _BPW)])
    pltpu.sync_copy(ib_v, ibg_hbm.at[pl.ds(base, _BPW)])


@jax.jit
def _sc_gather_dot(u_idx, i_idx, user_emb, item_emb, user_bias, item_bias):
    mesh = plsc.VectorSubcoreMesh(core_axis_name="c", subcore_axis_name="s")
    f = functools.partial(
        pl.kernel,
        mesh=mesh,
        out_type=(
            jax.ShapeDtypeStruct((_NW, _E), jnp.float32),
            jax.ShapeDtypeStruct((_B, 1), jnp.float32),
            jax.ShapeDtypeStruct((_B, 1), jnp.float32),
        ),
        scratch_types=[
            pltpu.VMEM((_BPW,), jnp.int32),
            pltpu.VMEM((_BPW,), jnp.int32),
            pltpu.VMEM((_BPW, _E), jnp.float32),
            pltpu.VMEM((_BPW, _E), jnp.float32),
            pltpu.VMEM((_BPW, 1), jnp.float32),
            pltpu.VMEM((_BPW, 1), jnp.float32),
            pltpu.VMEM((_E,), jnp.float32),
        ],
        compiler_params=pltpu.CompilerParams(use_tc_tiling_on_sc=False),
    )(_sc_body)
    return f(u_idx, i_idx, user_emb, item_emb, user_bias, item_bias)


def _tc_body(part_ref, ub_ref, ib_ref, out_ref):
    s = jnp.sum(part_ref[...])
    out_ref[...] = ub_ref[...] + ib_ref[...] + s


@jax.jit
def _tc_finish(partials, ubg, ibg):
    out = pl.pallas_call(
        _tc_body,
        out_shape=jax.ShapeDtypeStruct((128, 128), jnp.float32),
    )(partials, ubg.reshape(128, 128), ibg.reshape(128, 128))
    return out.reshape(_B, 1)


def kernel(inputs, user_emb, user_bias, item_emb, item_bias):
    u_idx = inputs[:, 0]
    i_idx = inputs[:, 1]
    partials, ubg, ibg = _sc_gather_dot(
        u_idx, i_idx, user_emb, item_emb, user_bias, item_bias)
    return _tc_finish(partials, ubg, ibg)


# speedup vs baseline: 3.0592x; 3.0592x over previous
import functools

import jax
import jax.numpy as jnp
from jax import lax
from jax.experimental import pallas as pl
from jax.experimental.pallas import tpu as pltpu
from jax.experimental.pallas import tpu_sc as plsc

_B = 16384
_E = 16
_V = 1000000
_NC = 2
_NS = 16
_NW = _NC * _NS
_BPW = _B // _NW
_CH = 128


def _sc_body(uidx_hbm, iidx_hbm, uemb_hbm, iemb_hbm, ubias_hbm, ibias_hbm,
             part_hbm, bsum_hbm,
             uidx_v, iidx_v, urows_v, irows_v, ub_v, ib_v, out_v, acc_v,
             sem, bsem):
    wid = lax.axis_index("s") * _NC + lax.axis_index("c")
    base = wid * _BPW

    pltpu.sync_copy(uidx_hbm.at[pl.ds(base, _BPW)], uidx_v)
    pltpu.sync_copy(iidx_hbm.at[pl.ds(base, _BPW)], iidx_v)

    handles = []
    for j in range(_BPW // _CH):
        c = pl.ds(j * _CH, _CH)
        handles.append(pltpu.async_copy(
            uemb_hbm.at[uidx_v.at[c]], urows_v.at[c], sem))
        handles.append(pltpu.async_copy(
            iemb_hbm.at[iidx_v.at[c]], irows_v.at[c], sem))
        handles.append(pltpu.async_copy(
            ubias_hbm.at[uidx_v.at[c]], ub_v.at[c], bsem))
        handles.append(pltpu.async_copy(
            ibias_hbm.at[iidx_v.at[c]], ib_v.at[c], bsem))
    for h in handles:
        h.wait()

    def fma(i, acc):
        return acc + urows_v[i] * irows_v[i]

    acc = lax.fori_loop(0, _BPW, fma, jnp.zeros((_E,), jnp.float32))
    acc_v[...] = acc
    pltpu.sync_copy(acc_v, part_hbm.at[wid])

    for j in range(_BPW // _E):
        s = pl.ds(j * _E, _E)
        out_v[s] = ub_v[s] + ib_v[s]
    pltpu.sync_copy(out_v, bsum_hbm.at[pl.ds(base, _BPW)])


@jax.jit
def _sc_parts(u_idx, i_idx, user_emb, item_emb, user_bias, item_bias):
    mesh = plsc.VectorSubcoreMesh(core_axis_name="c", subcore_axis_name="s")
    k = functools.partial(
        pl.kernel,
        mesh=mesh,
        out_type=[
            jax.ShapeDtypeStruct((_NW, _E), jnp.float32),
            jax.ShapeDtypeStruct((_B,), jnp.float32),
        ],
        scratch_types=[
            pltpu.VMEM((_BPW,), jnp.int32),
            pltpu.VMEM((_BPW,), jnp.int32),
            pltpu.VMEM((_BPW, _E), jnp.float32),
            pltpu.VMEM((_BPW, _E), jnp.float32),
            pltpu.VMEM((_BPW,), jnp.float32),
            pltpu.VMEM((_BPW,), jnp.float32),
            pltpu.VMEM((_BPW,), jnp.float32),
            pltpu.VMEM((_E,), jnp.float32),
            pltpu.SemaphoreType.DMA,
            pltpu.SemaphoreType.DMA,
        ],
        compiler_params=pltpu.CompilerParams(use_tc_tiling_on_sc=False),
    )(_sc_body)
    return k(u_idx, i_idx, user_emb, item_emb,
             user_bias.reshape(_V), item_bias.reshape(_V))


def _tc_body(part_ref, bs_ref, out_ref):
    s = jnp.sum(part_ref[...])
    out_ref[...] = bs_ref[...] + s


@jax.jit
def _tc_finish(partials, bias_sums):
    out = pl.pallas_call(
        _tc_body,
        out_shape=jax.ShapeDtypeStruct((128, 128), jnp.float32),
    )(partials, bias_sums.reshape(128, 128))
    return out.reshape(_B, 1)


def kernel(inputs, user_emb, user_bias, item_emb, item_bias):
    u_idx = inputs[:, 0]
    i_idx = inputs[:, 1]
    partials, bias_sums = _sc_parts(
        u_idx, i_idx, user_emb, item_emb, user_bias, item_bias)
    return _tc_finish(partials, bias_sums)
